# baseline (speedup 1.0000x reference)
import jax
import jax.numpy as jnp
from jax import lax
from jax.experimental import pallas as pl
from jax.experimental.pallas import tpu as pltpu
from jax.experimental.pallas import tpu_sc as plsc

N_FACTORS = 32
BATCH = 16384
NUM_WORKERS = 32
B_PER_W = BATCH // NUM_WORKERS
CHUNK = 128
NUM_CHUNKS = B_PER_W // CHUNK
LANES = 16
GROUPS_PER_CHUNK = CHUNK // LANES


def _sc_kernel(user_idx_hbm, item_idx_hbm, uf_hbm, if_hbm, out_hbm,
               idx_u, idx_v, u_buf, v_buf, out_v, sem_u, sem_v):
    cid = lax.axis_index("c")
    sid = lax.axis_index("s")
    wid = sid * 2 + cid
    base = wid * B_PER_W

    row_lane = lax.iota(jnp.int32, LANES)

    for c in range(NUM_CHUNKS):
        cbase = base + c * CHUNK
        pltpu.sync_copy(user_idx_hbm.at[pl.ds(cbase, CHUNK)], idx_u)
        pltpu.sync_copy(item_idx_hbm.at[pl.ds(cbase, CHUNK)], idx_v)
        cp_u = pltpu.async_copy(uf_hbm.at[idx_u], u_buf, sem_u)
        cp_v = pltpu.async_copy(if_hbm.at[idx_v], v_buf, sem_v)
        cp_u.wait()
        cp_v.wait()

        def group_body(g, c=c):
            rows = row_lane + g * LANES
            acc = jnp.zeros((LANES,), jnp.float32)
            for d in range(N_FACTORS):
                col = jnp.full((LANES,), d, jnp.int32)
                u_col = plsc.load_gather(u_buf, [rows, col])
                v_col = plsc.load_gather(v_buf, [rows, col])
                acc = acc + u_col * v_col
            sig = 1.0 / (1.0 + jnp.exp(-acc))
            out_v[pl.ds(c * CHUNK + g * LANES, LANES)] = sig

        pl.loop(0, GROUPS_PER_CHUNK)(group_body)

    pltpu.sync_copy(out_v, out_hbm.at[pl.ds(base, B_PER_W)])


@jax.jit
def kernel(X, user_factors, item_factors):
    user_idx = X[:, 0].astype(jnp.int32)
    item_idx = X[:, 1].astype(jnp.int32)

    mesh = plsc.VectorSubcoreMesh(core_axis_name="c", subcore_axis_name="s")
    run = pl.kernel(
        _sc_kernel,
        out_type=jax.ShapeDtypeStruct((BATCH,), jnp.float32),
        mesh=mesh,
        scratch_types=[
            pltpu.VMEM((CHUNK,), jnp.int32),
            pltpu.VMEM((CHUNK,), jnp.int32),
            pltpu.VMEM((CHUNK, N_FACTORS), jnp.float32),
            pltpu.VMEM((CHUNK, N_FACTORS), jnp.float32),
            pltpu.VMEM((B_PER_W,), jnp.float32),
            pltpu.SemaphoreType.DMA,
            pltpu.SemaphoreType.DMA,
        ],
        compiler_params=pltpu.CompilerParams(
            needs_layout_passes=False,
            use_tc_tiling_on_sc=False,
        ),
    )
    logits = run(user_idx, item_idx, user_factors, item_factors)
    return logits.reshape(BATCH, 1)
